# baseline (speedup 1.0000x reference)
import functools

import jax
import jax.numpy as jnp
from jax import lax
from jax.experimental import pallas as pl
from jax.experimental.pallas import tpu as pltpu
from jax.experimental.pallas import tpu_sc as plsc

_N, _D, _E = 10000, 128, 320000

_NC, _NS = 2, 16
_NW = _NC * _NS
_EPW = _E // _NW
_CHUNK = 80
_EPWP = 10240
_NCH = _EPWP // _CHUNK
_NP = 10240
_ROWS_PT = _NP // _NS


def _sc_agg_body(src_hbm, dst_hbm, x_hbm, zero_hbm, out_hbm,
                 si0, si1, si2, si3, di0, di1, di2, di3, rows0, rows1,
                 agg_sh, gsem0, gsem1, ssem0, ssem1, isem0, isem1):
    c = lax.axis_index("c")
    s = lax.axis_index("s")
    wid = s * _NC + c
    r0 = s * _ROWS_PT

    pltpu.sync_copy(zero_hbm.at[pl.ds(r0, _ROWS_PT)],
                    agg_sh.at[pl.ds(r0, _ROWS_PT)])
    plsc.subcore_barrier()

    rows = (rows0, rows1)
    gsem = (gsem0, gsem1)
    ssem = (ssem0, ssem1)
    isem = (isem0, isem1)
    sbufs = (si0, si1, si2, si3)
    dbufs = (di0, di1, di2, di3)
    base = wid * _EPWP

    def idx_fire(j, u, sm):
        pltpu.async_copy(src_hbm.at[pl.ds(base + j * _CHUNK, _CHUNK)],
                         sbufs[u], sm)
        pltpu.async_copy(dst_hbm.at[pl.ds(base + j * _CHUNK, _CHUNK)],
                         dbufs[u], sm)

    def idx_drain(sm):
        pltpu.make_async_copy(src_hbm.at[pl.ds(0, _CHUNK)], si0,
                              sm).wait()
        pltpu.make_async_copy(dst_hbm.at[pl.ds(0, _CHUNK)], di0,
                              sm).wait()

    def g_fire(u):
        pltpu.async_copy(x_hbm.at[sbufs[u % 4]], rows[u % 2],
                         gsem[u % 2])

    def g_drain(u):
        pltpu.make_async_copy(x_hbm.at[sbufs[u % 4]], rows[u % 2],
                              gsem[u % 2]).wait()

    def s_fire(u):
        pltpu.async_copy(rows[u % 2], agg_sh.at[dbufs[u % 4]],
                         ssem[u % 2], add=True)

    def s_drain(u):
        pltpu.make_async_copy(rows[u % 2], agg_sh.at[dbufs[u % 4]],
                              ssem[u % 2]).wait()

    def step(j, u, first=False, idx_ahead=True, gather_ahead=True):
        g_drain(u)
        s_fire(u)
        if not first:
            s_drain(u - 1)
        if idx_ahead:
            idx_fire(j + 2, (u + 2) % 4, isem[u % 2])
        if gather_ahead:
            idx_drain(isem[1 - u % 2])
            g_fire(u + 1)

    pltpu.sync_copy(src_hbm.at[pl.ds(base, _CHUNK)], si0)
    pltpu.sync_copy(dst_hbm.at[pl.ds(base, _CHUNK)], di0)
    idx_fire(1, 1, isem[1])
    g_fire(0)

    step(0, 0, first=True)
    step(1, 1)
    step(2, 2)
    step(3, 3)

    def outer(K, carry):
        j0 = 4 * K
        for u in range(4):
            step(j0 + u, u)
        return carry

    lax.fori_loop(1, _NCH // 4 - 1, outer, 0)
    step(124, 0)
    step(125, 1)
    step(126, 2, idx_ahead=False)
    step(127, 3, idx_ahead=False, gather_ahead=False)
    s_drain(3)
    plsc.subcore_barrier()

    pltpu.sync_copy(agg_sh.at[pl.ds(r0, _ROWS_PT)],
                    out_hbm.at[c, pl.ds(r0, _ROWS_PT)])


@functools.cache
def _sc_agg():
    return pl.kernel(
        _sc_agg_body,
        mesh=plsc.VectorSubcoreMesh(core_axis_name="c",
                                    subcore_axis_name="s",
                                    num_cores=_NC, num_subcores=_NS),
        out_type=jax.ShapeDtypeStruct((_NC, _NP, _D), jnp.float32),
        scratch_types=(
            [pltpu.VMEM((_CHUNK,), jnp.int32) for _ in range(8)]
            + [pltpu.VMEM((_CHUNK, _D), jnp.float32) for _ in range(2)]
            + [pltpu.VMEM_SHARED((_NP, _D), jnp.float32)]
            + [pltpu.SemaphoreType.DMA for _ in range(6)]
        ),
    )


_BR = 1000


def _mlp_body(eps_ref, x_ref, agg_ref, w1_ref, b1_ref, w2_ref, b2_ref,
              g_ref, bt_ref, o_ref):
    h = x_ref[...] * (1.0 + eps_ref[0]) + agg_ref[0] + agg_ref[1]
    h = jnp.dot(h, w1_ref[...], preferred_element_type=jnp.float32)
    h = h + b1_ref[...]
    h = 0.5 * h * (1.0 + lax.erf(h * 0.7071067811865476))
    h = jnp.dot(h, w2_ref[...], preferred_element_type=jnp.float32)
    h = h + b2_ref[...]
    m = jnp.mean(h, axis=-1, keepdims=True)
    v = jnp.mean(jnp.square(h - m), axis=-1, keepdims=True)
    o_ref[...] = (h - m) * lax.rsqrt(v + 1e-5) * g_ref[...] + bt_ref[...]


def _mlp(x, agg, w1, b1, w2, b2, gamma, beta, eps):
    grid = (_N // _BR,)
    return pl.pallas_call(
        _mlp_body,
        grid=grid,
        in_specs=[
            pl.BlockSpec(memory_space=pltpu.SMEM),
            pl.BlockSpec((_BR, _D), lambda i: (i, 0)),
            pl.BlockSpec((_NC, _BR, _D), lambda i: (0, i, 0)),
            pl.BlockSpec((_D, 2 * _D), lambda i: (0, 0)),
            pl.BlockSpec((1, 2 * _D), lambda i: (0, 0)),
            pl.BlockSpec((2 * _D, _D), lambda i: (0, 0)),
            pl.BlockSpec((1, _D), lambda i: (0, 0)),
            pl.BlockSpec((1, _D), lambda i: (0, 0)),
            pl.BlockSpec((1, _D), lambda i: (0, 0)),
        ],
        out_specs=pl.BlockSpec((_BR, _D), lambda i: (i, 0)),
        out_shape=jax.ShapeDtypeStruct((_N, _D), jnp.float32),
    )(eps, x, agg, w1, b1, w2, b2, gamma, beta)


def kernel(x, edge_index, W1, b1, W2, b2, eps, gamma, beta):
    pad = _EPWP - _EPW
    src = jnp.pad(edge_index[0].astype(jnp.int32).reshape(_NW, _EPW),
                  ((0, 0), (0, pad))).reshape(-1)
    dst = jnp.pad(edge_index[1].astype(jnp.int32).reshape(_NW, _EPW),
                  ((0, 0), (0, pad)),
                  constant_values=_N).reshape(-1)
    zeros = jnp.zeros((_NP, _D), jnp.float32)
    agg = _sc_agg()(src, dst, x, zeros)
    eps_arr = jnp.reshape(eps, (1,)).astype(jnp.float32)
    return _mlp(x, agg, W1, jnp.reshape(b1, (1, 2 * _D)), W2,
                jnp.reshape(b2, (1, _D)), jnp.reshape(gamma, (1, _D)),
                jnp.reshape(beta, (1, _D)), eps_arr)


# speedup vs baseline: 1.8966x; 1.8966x over previous
import functools

import jax
import jax.numpy as jnp
from jax import lax
from jax.experimental import pallas as pl
from jax.experimental.pallas import tpu as pltpu
from jax.experimental.pallas import tpu_sc as plsc

_N, _D, _E = 10000, 128, 320000

_NC, _NS = 2, 16
_NW = _NC * _NS
_EPW = _E // _NW
_CHUNK = 80
_NCH = _EPW // _CHUNK
_NP = 10240
_ROWS_PT = _NP // _NS


def _sc_agg_body(src_hbm, dst_hbm, x_hbm, zero_hbm, out_hbm,
                 src_v, dst_v, rows_v, agg_sh, gsem):
    c = lax.axis_index("c")
    s = lax.axis_index("s")
    wid = s * _NC + c
    r0 = s * _ROWS_PT

    pltpu.sync_copy(zero_hbm.at[pl.ds(r0, _ROWS_PT)],
                    agg_sh.at[pl.ds(r0, _ROWS_PT)])
    pltpu.sync_copy(src_hbm.at[wid], src_v)
    pltpu.sync_copy(dst_hbm.at[wid], dst_v)
    plsc.subcore_barrier()

    def body(j, carry):
        pltpu.async_copy(x_hbm.at[src_v.at[j]], rows_v, gsem).wait()
        pltpu.sync_copy(rows_v, agg_sh.at[dst_v.at[j]], add=True)
        return carry

    lax.fori_loop(0, _NCH, body, 0)
    plsc.subcore_barrier()

    pltpu.sync_copy(agg_sh.at[pl.ds(r0, _ROWS_PT)],
                    out_hbm.at[c, pl.ds(r0, _ROWS_PT)])


@functools.cache
def _sc_agg():
    return pl.kernel(
        _sc_agg_body,
        mesh=plsc.VectorSubcoreMesh(core_axis_name="c",
                                    subcore_axis_name="s",
                                    num_cores=_NC, num_subcores=_NS),
        out_type=jax.ShapeDtypeStruct((_NC, _NP, _D), jnp.float32),
        scratch_types=[
            pltpu.VMEM((_NCH, _CHUNK), jnp.int32),
            pltpu.VMEM((_NCH, _CHUNK), jnp.int32),
            pltpu.VMEM((_CHUNK, _D), jnp.float32),
            pltpu.VMEM_SHARED((_NP, _D), jnp.float32),
            pltpu.SemaphoreType.DMA,
        ],
    )


_BR = 1000


def _mlp_body(eps_ref, x_ref, agg_ref, w1_ref, b1_ref, w2_ref, b2_ref,
              g_ref, bt_ref, o_ref):
    h = x_ref[...] * (1.0 + eps_ref[0]) + agg_ref[0] + agg_ref[1]
    h = jnp.dot(h, w1_ref[...], preferred_element_type=jnp.float32)
    h = h + b1_ref[...]
    h = 0.5 * h * (1.0 + lax.erf(h * 0.7071067811865476))
    h = jnp.dot(h, w2_ref[...], preferred_element_type=jnp.float32)
    h = h + b2_ref[...]
    m = jnp.mean(h, axis=-1, keepdims=True)
    v = jnp.mean(jnp.square(h - m), axis=-1, keepdims=True)
    o_ref[...] = (h - m) * lax.rsqrt(v + 1e-5) * g_ref[...] + bt_ref[...]


def _mlp(x, agg, w1, b1, w2, b2, gamma, beta, eps):
    grid = (_N // _BR,)
    return pl.pallas_call(
        _mlp_body,
        grid=grid,
        in_specs=[
            pl.BlockSpec(memory_space=pltpu.SMEM),
            pl.BlockSpec((_BR, _D), lambda i: (i, 0)),
            pl.BlockSpec((_NC, _BR, _D), lambda i: (0, i, 0)),
            pl.BlockSpec((_D, 2 * _D), lambda i: (0, 0)),
            pl.BlockSpec((1, 2 * _D), lambda i: (0, 0)),
            pl.BlockSpec((2 * _D, _D), lambda i: (0, 0)),
            pl.BlockSpec((1, _D), lambda i: (0, 0)),
            pl.BlockSpec((1, _D), lambda i: (0, 0)),
            pl.BlockSpec((1, _D), lambda i: (0, 0)),
        ],
        out_specs=pl.BlockSpec((_BR, _D), lambda i: (i, 0)),
        out_shape=jax.ShapeDtypeStruct((_N, _D), jnp.float32),
    )(eps, x, agg, w1, b1, w2, b2, gamma, beta)


def kernel(x, edge_index, W1, b1, W2, b2, eps, gamma, beta):
    src = edge_index[0].astype(jnp.int32).reshape(_NW, _NCH, _CHUNK)
    dst = edge_index[1].astype(jnp.int32).reshape(_NW, _NCH, _CHUNK)
    zeros = jnp.zeros((_NP, _D), jnp.float32)
    agg = _sc_agg()(src, dst, x, zeros)
    eps_arr = jnp.reshape(eps, (1,)).astype(jnp.float32)
    return _mlp(x, agg, W1, jnp.reshape(b1, (1, 2 * _D)), W2,
                jnp.reshape(b2, (1, _D)), jnp.reshape(gamma, (1, _D)),
                jnp.reshape(beta, (1, _D)), eps_arr)


# speedup vs baseline: 2.0411x; 1.0762x over previous
import functools

import jax
import jax.numpy as jnp
from jax import lax
from jax.experimental import pallas as pl
from jax.experimental.pallas import tpu as pltpu
from jax.experimental.pallas import tpu_sc as plsc

_N, _D, _E = 10000, 128, 320000

_NC, _NS = 2, 16
_NW = _NC * _NS
_EPW = _E // _NW
_CHUNK = 100
_NCH = _EPW // _CHUNK
_NP = 10240
_ROWS_PT = _NP // _NS


def _sc_agg_body(src_hbm, dst_hbm, x_hbm, zero_hbm, out_hbm,
                 src_v, dst_v, rows_v, agg_sh, gsem):
    c = lax.axis_index("c")
    s = lax.axis_index("s")
    wid = s * _NC + c
    r0 = s * _ROWS_PT

    pltpu.sync_copy(zero_hbm.at[pl.ds(r0, _ROWS_PT)],
                    agg_sh.at[pl.ds(r0, _ROWS_PT)])
    pltpu.sync_copy(src_hbm.at[wid], src_v)
    pltpu.sync_copy(dst_hbm.at[wid], dst_v)
    plsc.subcore_barrier()

    def body(j, carry):
        pltpu.async_copy(x_hbm.at[src_v.at[j]], rows_v, gsem).wait()
        pltpu.sync_copy(rows_v, agg_sh.at[dst_v.at[j]], add=True)
        return carry

    lax.fori_loop(0, _NCH, body, 0)
    plsc.subcore_barrier()

    pltpu.sync_copy(agg_sh.at[pl.ds(r0, _ROWS_PT)],
                    out_hbm.at[c, pl.ds(r0, _ROWS_PT)])


@functools.cache
def _sc_agg():
    return pl.kernel(
        _sc_agg_body,
        mesh=plsc.VectorSubcoreMesh(core_axis_name="c",
                                    subcore_axis_name="s",
                                    num_cores=_NC, num_subcores=_NS),
        out_type=jax.ShapeDtypeStruct((_NC, _NP, _D), jnp.float32),
        scratch_types=[
            pltpu.VMEM((_NCH, _CHUNK), jnp.int32),
            pltpu.VMEM((_NCH, _CHUNK), jnp.int32),
            pltpu.VMEM((_CHUNK, _D), jnp.float32),
            pltpu.VMEM_SHARED((_NP, _D), jnp.float32),
            pltpu.SemaphoreType.DMA,
        ],
    )


_BR = 1000


def _mlp_body(eps_ref, x_ref, agg_ref, w1_ref, b1_ref, w2_ref, b2_ref,
              g_ref, bt_ref, o_ref):
    h = x_ref[...] * (1.0 + eps_ref[0]) + agg_ref[0] + agg_ref[1]
    h = jnp.dot(h, w1_ref[...], preferred_element_type=jnp.float32)
    h = h + b1_ref[...]
    h = 0.5 * h * (1.0 + lax.erf(h * 0.7071067811865476))
    h = jnp.dot(h, w2_ref[...], preferred_element_type=jnp.float32)
    h = h + b2_ref[...]
    m = jnp.mean(h, axis=-1, keepdims=True)
    v = jnp.mean(jnp.square(h - m), axis=-1, keepdims=True)
    o_ref[...] = (h - m) * lax.rsqrt(v + 1e-5) * g_ref[...] + bt_ref[...]


def _mlp(x, agg, w1, b1, w2, b2, gamma, beta, eps):
    grid = (_N // _BR,)
    return pl.pallas_call(
        _mlp_body,
        grid=grid,
        in_specs=[
            pl.BlockSpec(memory_space=pltpu.SMEM),
            pl.BlockSpec((_BR, _D), lambda i: (i, 0)),
            pl.BlockSpec((_NC, _BR, _D), lambda i: (0, i, 0)),
            pl.BlockSpec((_D, 2 * _D), lambda i: (0, 0)),
            pl.BlockSpec((1, 2 * _D), lambda i: (0, 0)),
            pl.BlockSpec((2 * _D, _D), lambda i: (0, 0)),
            pl.BlockSpec((1, _D), lambda i: (0, 0)),
            pl.BlockSpec((1, _D), lambda i: (0, 0)),
            pl.BlockSpec((1, _D), lambda i: (0, 0)),
        ],
        out_specs=pl.BlockSpec((_BR, _D), lambda i: (i, 0)),
        out_shape=jax.ShapeDtypeStruct((_N, _D), jnp.float32),
    )(eps, x, agg, w1, b1, w2, b2, gamma, beta)


def kernel(x, edge_index, W1, b1, W2, b2, eps, gamma, beta):
    src = edge_index[0].astype(jnp.int32).reshape(_NW, _NCH, _CHUNK)
    dst = edge_index[1].astype(jnp.int32).reshape(_NW, _NCH, _CHUNK)
    zeros = jnp.zeros((_NP, _D), jnp.float32)
    agg = _sc_agg()(src, dst, x, zeros)
    eps_arr = jnp.reshape(eps, (1,)).astype(jnp.float32)
    return _mlp(x, agg, W1, jnp.reshape(b1, (1, 2 * _D)), W2,
                jnp.reshape(b2, (1, _D)), jnp.reshape(gamma, (1, _D)),
                jnp.reshape(beta, (1, _D)), eps_arr)


# speedup vs baseline: 2.1391x; 1.0480x over previous
import functools

import jax
import jax.numpy as jnp
from jax import lax
from jax.experimental import pallas as pl
from jax.experimental.pallas import tpu as pltpu
from jax.experimental.pallas import tpu_sc as plsc

_N, _D, _E = 10000, 128, 320000

_NC, _NS = 2, 16
_NW = _NC * _NS
_EPW = _E // _NW
_CHUNK = 125
_NCH = _EPW // _CHUNK
_NP = 10240
_ROWS_PT = _NP // _NS


def _sc_agg_body(src_hbm, dst_hbm, x_hbm, zero_hbm, out_hbm,
                 src_v, dst_v, rows_v, agg_sh, gsem):
    c = lax.axis_index("c")
    s = lax.axis_index("s")
    wid = s * _NC + c
    r0 = s * _ROWS_PT

    pltpu.sync_copy(zero_hbm.at[pl.ds(r0, _ROWS_PT)],
                    agg_sh.at[pl.ds(r0, _ROWS_PT)])
    pltpu.sync_copy(src_hbm.at[wid], src_v)
    pltpu.sync_copy(dst_hbm.at[wid], dst_v)
    plsc.subcore_barrier()

    def body(j, carry):
        pltpu.async_copy(x_hbm.at[src_v.at[j]], rows_v, gsem).wait()
        pltpu.sync_copy(rows_v, agg_sh.at[dst_v.at[j]], add=True)
        return carry

    lax.fori_loop(0, _NCH, body, 0)
    plsc.subcore_barrier()

    pltpu.sync_copy(agg_sh.at[pl.ds(r0, _ROWS_PT)],
                    out_hbm.at[c, pl.ds(r0, _ROWS_PT)])


@functools.cache
def _sc_agg():
    return pl.kernel(
        _sc_agg_body,
        mesh=plsc.VectorSubcoreMesh(core_axis_name="c",
                                    subcore_axis_name="s",
                                    num_cores=_NC, num_subcores=_NS),
        out_type=jax.ShapeDtypeStruct((_NC, _NP, _D), jnp.float32),
        scratch_types=[
            pltpu.VMEM((_NCH, _CHUNK), jnp.int32),
            pltpu.VMEM((_NCH, _CHUNK), jnp.int32),
            pltpu.VMEM((_CHUNK, _D), jnp.float32),
            pltpu.VMEM_SHARED((_NP, _D), jnp.float32),
            pltpu.SemaphoreType.DMA,
        ],
    )


_BR = 1000


def _mlp_body(eps_ref, x_ref, agg_ref, w1_ref, b1_ref, w2_ref, b2_ref,
              g_ref, bt_ref, o_ref):
    h = x_ref[...] * (1.0 + eps_ref[0]) + agg_ref[0] + agg_ref[1]
    h = jnp.dot(h, w1_ref[...], preferred_element_type=jnp.float32)
    h = h + b1_ref[...]
    h = 0.5 * h * (1.0 + lax.erf(h * 0.7071067811865476))
    h = jnp.dot(h, w2_ref[...], preferred_element_type=jnp.float32)
    h = h + b2_ref[...]
    m = jnp.mean(h, axis=-1, keepdims=True)
    v = jnp.mean(jnp.square(h - m), axis=-1, keepdims=True)
    o_ref[...] = (h - m) * lax.rsqrt(v + 1e-5) * g_ref[...] + bt_ref[...]


def _mlp(x, agg, w1, b1, w2, b2, gamma, beta, eps):
    grid = (_N // _BR,)
    return pl.pallas_call(
        _mlp_body,
        grid=grid,
        in_specs=[
            pl.BlockSpec(memory_space=pltpu.SMEM),
            pl.BlockSpec((_BR, _D), lambda i: (i, 0)),
            pl.BlockSpec((_NC, _BR, _D), lambda i: (0, i, 0)),
            pl.BlockSpec((_D, 2 * _D), lambda i: (0, 0)),
            pl.BlockSpec((1, 2 * _D), lambda i: (0, 0)),
            pl.BlockSpec((2 * _D, _D), lambda i: (0, 0)),
            pl.BlockSpec((1, _D), lambda i: (0, 0)),
            pl.BlockSpec((1, _D), lambda i: (0, 0)),
            pl.BlockSpec((1, _D), lambda i: (0, 0)),
        ],
        out_specs=pl.BlockSpec((_BR, _D), lambda i: (i, 0)),
        out_shape=jax.ShapeDtypeStruct((_N, _D), jnp.float32),
    )(eps, x, agg, w1, b1, w2, b2, gamma, beta)


def kernel(x, edge_index, W1, b1, W2, b2, eps, gamma, beta):
    src = edge_index[0].astype(jnp.int32).reshape(_NW, _NCH, _CHUNK)
    dst = edge_index[1].astype(jnp.int32).reshape(_NW, _NCH, _CHUNK)
    zeros = jnp.zeros((_NP, _D), jnp.float32)
    agg = _sc_agg()(src, dst, x, zeros)
    eps_arr = jnp.reshape(eps, (1,)).astype(jnp.float32)
    return _mlp(x, agg, W1, jnp.reshape(b1, (1, 2 * _D)), W2,
                jnp.reshape(b2, (1, _D)), jnp.reshape(gamma, (1, _D)),
                jnp.reshape(beta, (1, _D)), eps_arr)
